# baseline (speedup 1.0000x reference)
import dataclasses
import functools

import jax
import jax.numpy as jnp
from jax import lax
from jax.experimental import pallas as pl
from jax.experimental.pallas import tpu as pltpu
from jax.experimental.pallas import tpu_sc as plsc

_N = 100000
_E = 1600000
_ALPHA = 0.85
_ITERS = 100
_NSRC = 256

_LANES = 16
_TILES = 16
_BLK = 2048
_NBLK = 52
_EPT = _BLK * _NBLK
_EPAD = _EPT * _TILES
_NVIRT = 512
_NPAD = 100096
_NSLICE = _NPAD // _TILES
_BINIT = (1.0 - _ALPHA) / (_ALPHA * _NSRC)
_WSELF = 1.0 / _ALPHA
_CHUNKS = ((0, 2048), (2048, 2048), (4096, 2048), (6144, 112))


def _f32x16(v):
    return jnp.full((_LANES,), v, dtype=jnp.float32)


def _body(src_hbm, dst_hbm, ew_hbm, xout_hbm, w_hbm,
          x_vmem, srcv, wv, pv, srcv1, wv1, pv1,
          dstv, dstv1, dstv2, dstv3, zbuf,
          xmsh, sem_in0, sem_in1, sem_sc0, sem_sc1):
    sid = lax.axis_index("s")
    ebase = sid * _EPT
    nbase = sid * _NSLICE

    sw = ((srcv, wv, pv, sem_in0, sem_sc0),
          (srcv1, wv1, pv1, sem_in1, sem_sc1))
    dbufs = (dstv, dstv1, dstv2, dstv3)

    def issue_in(b, m):
        sb, wb, _, sem, _ = sw[m % 2]
        e0 = b * _BLK + ebase
        pltpu.async_copy(src_hbm.at[pl.ds(e0, _BLK)], sb, sem)
        pltpu.async_copy(dst_hbm.at[pl.ds(e0, _BLK)], dbufs[m], sem)
        pltpu.async_copy(w_hbm.at[pl.ds(e0, _BLK)], wb, sem)

    def wait_in(m):
        sb, wb, _, sem, _ = sw[m % 2]
        pltpu.make_async_copy(src_hbm.at[pl.ds(0, _BLK)], sb, sem).wait()
        pltpu.make_async_copy(dst_hbm.at[pl.ds(0, _BLK)], dbufs[m],
                              sem).wait()
        pltpu.make_async_copy(w_hbm.at[pl.ds(0, _BLK)], wb, sem).wait()

    def compute(m):
        sb, wb, pb, _, _ = sw[m % 2]
        for i in range(_BLK // _LANES):
            sl = pl.ds(i * _LANES, _LANES)
            xg = plsc.load_gather(x_vmem, [sb[sl]])
            pb[sl] = xg * wb[sl]

    def issue_sc(m):
        _, _, pb, _, sem = sw[m % 2]
        pltpu.async_copy(pb, xmsh.at[dbufs[m]], sem, add=True)

    def wait_sc(m):
        _, _, pb, _, sem = sw[m % 2]
        pltpu.make_async_copy(pb, xmsh.at[dbufs[m]], sem).wait()

    @pl.loop(0, _BLK // _LANES)
    def _(j):
        zbuf[pl.ds(j * _LANES, _LANES)] = _f32x16(0.0)

    for off, ln in _CHUNKS:
        pltpu.sync_copy(zbuf.at[pl.ds(0, ln)],
                        xmsh.at[pl.ds(nbase + off, ln)])
    plsc.subcore_barrier()

    @pl.loop(0, _NBLK)
    def _(blk):
        e0 = ebase + blk * _BLK
        pltpu.sync_copy(src_hbm.at[pl.ds(e0, _BLK)], srcv)
        pltpu.sync_copy(ew_hbm.at[pl.ds(e0, _BLK)], wv)
        pltpu.sync_copy(wv, xmsh.at[srcv], add=True)
    plsc.subcore_barrier()

    pltpu.sync_copy(xmsh, x_vmem)
    plsc.subcore_barrier()

    for off, ln in _CHUNKS:
        pltpu.sync_copy(zbuf.at[pl.ds(0, ln)],
                        xmsh.at[pl.ds(nbase + off, ln)])

    @pl.loop(0, _NBLK)
    def _(blk):
        e0 = ebase + blk * _BLK
        pltpu.sync_copy(src_hbm.at[pl.ds(e0, _BLK)], srcv)
        pltpu.sync_copy(ew_hbm.at[pl.ds(e0, _BLK)], wv)
        for i in range(_BLK // _LANES):
            sl = pl.ds(i * _LANES, _LANES)
            rs = plsc.load_gather(x_vmem, [srcv[sl]])
            nz = rs != _f32x16(0.0)
            safe = jnp.where(nz, rs, _f32x16(1.0))
            pv[sl] = jnp.where(nz, wv[sl] / safe, _f32x16(0.0))
        pltpu.sync_copy(pv, w_hbm.at[pl.ds(e0, _BLK)])

    @pl.when(sid == 0)
    def _():
        lane0 = lax.iota(jnp.int32, _LANES) == 0
        pv[pl.ds(0, _LANES)] = jnp.where(lane0, _f32x16(_WSELF),
                                         _f32x16(_BINIT))
        for k in range(1, _NSRC // _LANES):
            pv[pl.ds(k * _LANES, _LANES)] = _f32x16(_BINIT)
        pv[pl.ds(_NSRC, _LANES)] = jnp.where(lane0, _f32x16(_BINIT),
                                             _f32x16(0.0))
        for k in range(_NSRC // _LANES + 1, _NVIRT // _LANES):
            pv[pl.ds(k * _LANES, _LANES)] = _f32x16(0.0)
        pltpu.sync_copy(pv.at[pl.ds(0, _NVIRT)], w_hbm.at[pl.ds(0, _NVIRT)])

    @pl.loop(0, _NPAD // _LANES)
    def _(j):
        x_vmem[pl.ds(j * _LANES, _LANES)] = _f32x16(1.0 / _N)
    x_vmem[pl.ds(_N, _LANES)] = jnp.where(lax.iota(jnp.int32, _LANES) == 0,
                                          _f32x16(1.0), _f32x16(0.0))
    plsc.subcore_barrier()

    @pl.loop(0, _ITERS)
    def _(it):
        issue_in(0, 0)
        issue_in(1, 1)

        @pl.loop(0, _NBLK // 4)
        def _(g):
            b0 = 4 * g
            for j in range(4):
                wait_in(j)
                compute(j)
                _, _, pb, _, _ = sw[j % 2]
                pltpu.sync_copy(pb, xmsh.at[dbufs[j]], add=True)

                @pl.when(b0 + j + 2 < _NBLK)
                def _():
                    issue_in(b0 + j + 2, (j + 2) % 4)
        plsc.subcore_barrier()

        for off, ln in _CHUNKS:
            pltpu.sync_copy(xmsh.at[pl.ds(nbase + off, ln)],
                            pv.at[pl.ds(0, ln)])

            @pl.loop(0, ln // _LANES)
            def _(j):
                sl = pl.ds(j * _LANES, _LANES)
                pv[sl] = pv[sl] * _ALPHA

            pltpu.sync_copy(pv.at[pl.ds(0, ln)],
                            xout_hbm.at[pl.ds(nbase + off, ln)])
            pltpu.sync_copy(zbuf.at[pl.ds(0, ln)],
                            xmsh.at[pl.ds(nbase + off, ln)])
        plsc.subcore_barrier()
        pltpu.sync_copy(xout_hbm, x_vmem)


_cp = pltpu.CompilerParams()
if "needs_layout_passes" in pltpu.CompilerParams.__dataclass_fields__:
    _cp = dataclasses.replace(_cp, needs_layout_passes=False)

_pr_call = functools.partial(
    pl.kernel,
    compiler_params=_cp,
    out_type=(jax.ShapeDtypeStruct((_NPAD,), jnp.float32),
              jax.ShapeDtypeStruct((_EPAD,), jnp.float32)),
    mesh=plsc.VectorSubcoreMesh(core_axis_name="c", subcore_axis_name="s",
                                num_cores=1),
    scratch_types=[
        pltpu.VMEM((_NPAD,), jnp.float32),
        pltpu.VMEM((_BLK,), jnp.int32),
        pltpu.VMEM((_BLK,), jnp.float32),
        pltpu.VMEM((_BLK,), jnp.float32),
        pltpu.VMEM((_BLK,), jnp.int32),
        pltpu.VMEM((_BLK,), jnp.float32),
        pltpu.VMEM((_BLK,), jnp.float32),
        pltpu.VMEM((_BLK,), jnp.int32),
        pltpu.VMEM((_BLK,), jnp.int32),
        pltpu.VMEM((_BLK,), jnp.int32),
        pltpu.VMEM((_BLK,), jnp.int32),
        pltpu.VMEM((_BLK,), jnp.float32),
        pltpu.VMEM_SHARED((_NPAD,), jnp.float32),
        pltpu.SemaphoreType.DMA,
        pltpu.SemaphoreType.DMA,
        pltpu.SemaphoreType.DMA,
        pltpu.SemaphoreType.DMA,
    ],
)(_body)


def kernel(edge_index, edge_weight, source_nodes):
    src = edge_index[0]
    dst = edge_index[1]
    pad = _EPAD - _E - _NVIRT
    fill = jnp.arange(pad, dtype=jnp.int32) % _N
    vsrc = jnp.full((_NVIRT,), _N, dtype=jnp.int32)
    vdst = jnp.concatenate([
        jnp.full((1,), _N, dtype=jnp.int32),
        source_nodes.astype(jnp.int32),
        jnp.arange(_NVIRT - 1 - _NSRC, dtype=jnp.int32) % _N,
    ])
    vw = jnp.zeros((_NVIRT,), jnp.float32)
    src1 = jnp.concatenate([vsrc, src, fill])
    dst1 = jnp.concatenate([vdst, dst, fill])
    ew1 = jnp.concatenate([vw, edge_weight, jnp.zeros((pad,), jnp.float32)])
    xpad, _ = _pr_call(src1, dst1, ew1)
    return xpad[:_N]


# speedup vs baseline: 1.3377x; 1.3377x over previous
import dataclasses
import functools

import jax
import jax.numpy as jnp
from jax import lax
from jax.experimental import pallas as pl
from jax.experimental.pallas import tpu as pltpu
from jax.experimental.pallas import tpu_sc as plsc

_N = 100000
_E = 1600000
_ALPHA = 0.85
_ITERS = 100
_NSRC = 256

_LANES = 16
_TILES = 16
_BLK = 2048
_NBLK = 52
_EPT = _BLK * _NBLK
_EPAD = _EPT * _TILES
_NVIRT = 512
_NPAD = 100096
_NSLICE = _NPAD // _TILES
_BINIT = (1.0 - _ALPHA) / (_ALPHA * _NSRC)
_WSELF = 1.0 / _ALPHA
_CHUNKS = ((0, 2048), (2048, 2048), (4096, 2048), (6144, 112))


def _f32x16(v):
    return jnp.full((_LANES,), v, dtype=jnp.float32)


def _body(src_hbm, dst_hbm, ew_hbm, xout_hbm, w_hbm,
          x_vmem, srcv, wv, pv, srcv1, wv1, pv1,
          dstv, dstv1, dstv2, dstv3, zbuf,
          xmsh, sem_in0, sem_in1, sem_sc0, sem_sc1):
    sid = lax.axis_index("s")
    ebase = sid * _EPT
    nbase = sid * _NSLICE

    sw = ((srcv, wv, pv, sem_in0, sem_sc0),
          (srcv1, wv1, pv1, sem_in1, sem_sc1))
    dbufs = (dstv, dstv1, dstv2, dstv3)

    def issue_in(b, m):
        sb, wb, _, sem, _ = sw[m % 2]
        e0 = b * _BLK + ebase
        pltpu.async_copy(src_hbm.at[pl.ds(e0, _BLK)], sb, sem)
        pltpu.async_copy(dst_hbm.at[pl.ds(e0, _BLK)], dbufs[m], sem)
        pltpu.async_copy(w_hbm.at[pl.ds(e0, _BLK)], wb, sem)

    def wait_in(m):
        sb, wb, _, sem, _ = sw[m % 2]
        pltpu.make_async_copy(src_hbm.at[pl.ds(0, _BLK)], sb, sem).wait()
        pltpu.make_async_copy(dst_hbm.at[pl.ds(0, _BLK)], dbufs[m],
                              sem).wait()
        pltpu.make_async_copy(w_hbm.at[pl.ds(0, _BLK)], wb, sem).wait()

    def compute(m):
        sb, wb, pb, _, _ = sw[m % 2]
        for i in range(_BLK // _LANES):
            sl = pl.ds(i * _LANES, _LANES)
            xg = plsc.load_gather(x_vmem, [sb[sl]])
            pb[sl] = xg * wb[sl]

    def issue_sc(m):
        _, _, pb, _, sem = sw[m % 2]
        pltpu.async_copy(pb, xmsh.at[dbufs[m]], sem, add=True)

    def wait_sc(m):
        _, _, pb, _, sem = sw[m % 2]
        pltpu.make_async_copy(pb, xmsh.at[dbufs[m]], sem).wait()

    @pl.loop(0, _BLK // _LANES)
    def _(j):
        zbuf[pl.ds(j * _LANES, _LANES)] = _f32x16(0.0)

    for off, ln in _CHUNKS:
        pltpu.sync_copy(zbuf.at[pl.ds(0, ln)],
                        xmsh.at[pl.ds(nbase + off, ln)])
    plsc.subcore_barrier()

    @pl.loop(0, _NBLK)
    def _(blk):
        e0 = ebase + blk * _BLK
        pltpu.sync_copy(src_hbm.at[pl.ds(e0, _BLK)], srcv)
        pltpu.sync_copy(ew_hbm.at[pl.ds(e0, _BLK)], wv)
        pltpu.sync_copy(wv, xmsh.at[srcv], add=True)
    plsc.subcore_barrier()

    pltpu.sync_copy(xmsh, x_vmem)
    plsc.subcore_barrier()

    for off, ln in _CHUNKS:
        pltpu.sync_copy(zbuf.at[pl.ds(0, ln)],
                        xmsh.at[pl.ds(nbase + off, ln)])

    @pl.loop(0, _NBLK)
    def _(blk):
        e0 = ebase + blk * _BLK
        pltpu.sync_copy(src_hbm.at[pl.ds(e0, _BLK)], srcv)
        pltpu.sync_copy(ew_hbm.at[pl.ds(e0, _BLK)], wv)
        for i in range(_BLK // _LANES):
            sl = pl.ds(i * _LANES, _LANES)
            rs = plsc.load_gather(x_vmem, [srcv[sl]])
            nz = rs != _f32x16(0.0)
            safe = jnp.where(nz, rs, _f32x16(1.0))
            pv[sl] = jnp.where(nz, wv[sl] / safe, _f32x16(0.0))
        pltpu.sync_copy(pv, w_hbm.at[pl.ds(e0, _BLK)])

    @pl.when(sid == 0)
    def _():
        lane0 = lax.iota(jnp.int32, _LANES) == 0
        pv[pl.ds(0, _LANES)] = jnp.where(lane0, _f32x16(_WSELF),
                                         _f32x16(_BINIT))
        for k in range(1, _NSRC // _LANES):
            pv[pl.ds(k * _LANES, _LANES)] = _f32x16(_BINIT)
        pv[pl.ds(_NSRC, _LANES)] = jnp.where(lane0, _f32x16(_BINIT),
                                             _f32x16(0.0))
        for k in range(_NSRC // _LANES + 1, _NVIRT // _LANES):
            pv[pl.ds(k * _LANES, _LANES)] = _f32x16(0.0)
        pltpu.sync_copy(pv.at[pl.ds(0, _NVIRT)], w_hbm.at[pl.ds(0, _NVIRT)])

    @pl.loop(0, _NPAD // _LANES)
    def _(j):
        x_vmem[pl.ds(j * _LANES, _LANES)] = _f32x16(1.0 / _N)
    x_vmem[pl.ds(_N, _LANES)] = jnp.where(lax.iota(jnp.int32, _LANES) == 0,
                                          _f32x16(1.0), _f32x16(0.0))
    plsc.subcore_barrier()

    @pl.loop(0, _ITERS)
    def _(it):
        issue_in(0, 0)
        issue_in(1, 1)

        @pl.loop(0, _NBLK // 4)
        def _(g):
            b0 = 4 * g
            for j in range(4):
                wait_in(j)
                compute(j)
                if j == 0:
                    @pl.when(g > 0)
                    def _():
                        wait_sc(3)
                else:
                    wait_sc(j - 1)
                issue_sc(j)

                @pl.when(b0 + j + 2 < _NBLK)
                def _():
                    issue_in(b0 + j + 2, (j + 2) % 4)

        wait_sc(3)
        pltpu.sync_copy(zbuf, xmsh.at[dbufs[3]], add=True)
        plsc.subcore_barrier()

        for off, ln in _CHUNKS:
            pltpu.sync_copy(xmsh.at[pl.ds(nbase + off, ln)],
                            pv.at[pl.ds(0, ln)])

            @pl.loop(0, ln // _LANES)
            def _(j):
                sl = pl.ds(j * _LANES, _LANES)
                pv[sl] = pv[sl] * _ALPHA

            pltpu.sync_copy(pv.at[pl.ds(0, ln)],
                            xout_hbm.at[pl.ds(nbase + off, ln)])
            pltpu.sync_copy(zbuf.at[pl.ds(0, ln)],
                            xmsh.at[pl.ds(nbase + off, ln)])
        plsc.subcore_barrier()
        pltpu.sync_copy(xout_hbm, x_vmem)


_cp = pltpu.CompilerParams()
if "needs_layout_passes" in pltpu.CompilerParams.__dataclass_fields__:
    _cp = dataclasses.replace(_cp, needs_layout_passes=False)

_pr_call = functools.partial(
    pl.kernel,
    compiler_params=_cp,
    out_type=(jax.ShapeDtypeStruct((_NPAD,), jnp.float32),
              jax.ShapeDtypeStruct((_EPAD,), jnp.float32)),
    mesh=plsc.VectorSubcoreMesh(core_axis_name="c", subcore_axis_name="s",
                                num_cores=1),
    scratch_types=[
        pltpu.VMEM((_NPAD,), jnp.float32),
        pltpu.VMEM((_BLK,), jnp.int32),
        pltpu.VMEM((_BLK,), jnp.float32),
        pltpu.VMEM((_BLK,), jnp.float32),
        pltpu.VMEM((_BLK,), jnp.int32),
        pltpu.VMEM((_BLK,), jnp.float32),
        pltpu.VMEM((_BLK,), jnp.float32),
        pltpu.VMEM((_BLK,), jnp.int32),
        pltpu.VMEM((_BLK,), jnp.int32),
        pltpu.VMEM((_BLK,), jnp.int32),
        pltpu.VMEM((_BLK,), jnp.int32),
        pltpu.VMEM((_BLK,), jnp.float32),
        pltpu.VMEM_SHARED((_NPAD,), jnp.float32),
        pltpu.SemaphoreType.DMA,
        pltpu.SemaphoreType.DMA,
        pltpu.SemaphoreType.DMA,
        pltpu.SemaphoreType.DMA,
    ],
)(_body)


def kernel(edge_index, edge_weight, source_nodes):
    src = edge_index[0]
    dst = edge_index[1]
    pad = _EPAD - _E - _NVIRT
    fill = jnp.arange(pad, dtype=jnp.int32) % _N
    vsrc = jnp.full((_NVIRT,), _N, dtype=jnp.int32)
    vdst = jnp.concatenate([
        jnp.full((1,), _N, dtype=jnp.int32),
        source_nodes.astype(jnp.int32),
        jnp.arange(_NVIRT - 1 - _NSRC, dtype=jnp.int32) % _N,
    ])
    vw = jnp.zeros((_NVIRT,), jnp.float32)
    src1 = jnp.concatenate([vsrc, src, fill])
    dst1 = jnp.concatenate([vdst, dst, fill])
    ew1 = jnp.concatenate([vw, edge_weight, jnp.zeros((pad,), jnp.float32)])
    xpad, _ = _pr_call(src1, dst1, ew1)
    return xpad[:_N]


# speedup vs baseline: 1.3521x; 1.0108x over previous
import dataclasses
import functools

import jax
import jax.numpy as jnp
from jax import lax
from jax.experimental import pallas as pl
from jax.experimental.pallas import tpu as pltpu
from jax.experimental.pallas import tpu_sc as plsc

_N = 100000
_E = 1600000
_ALPHA = 0.85
_ITERS = 100
_NSRC = 256

_LANES = 16
_TILES = 16
_BLK = 2048
_NBLK = 52
_EPT = _BLK * _NBLK
_EPAD = _EPT * _TILES
_NVIRT = 512
_NPAD = 100096
_NSLICE = _NPAD // _TILES
_BINIT = (1.0 - _ALPHA) / (_ALPHA * _NSRC)
_WSELF = 1.0 / _ALPHA
_CHUNKS = ((0, 2048), (2048, 2048), (4096, 2048), (6144, 112))


def _f32x16(v):
    return jnp.full((_LANES,), v, dtype=jnp.float32)


def _body(src_hbm, dst_hbm, ew_hbm, xout_hbm, w_hbm,
          x_vmem, srcv, wv, pv, srcv1, wv1, pv1,
          dstv, dstv1, dstv2, dstv3, zbuf,
          xmsh, sem_in0, sem_in1, sem_sc0, sem_sc1, sem_bc):
    sid = lax.axis_index("s")
    ebase = sid * _EPT
    nbase = sid * _NSLICE

    sw = ((srcv, wv, pv, sem_in0, sem_sc0),
          (srcv1, wv1, pv1, sem_in1, sem_sc1))
    dbufs = (dstv, dstv1, dstv2, dstv3)

    def issue_in(b, m):
        sb, wb, _, sem, _ = sw[m % 2]
        e0 = b * _BLK + ebase
        pltpu.async_copy(src_hbm.at[pl.ds(e0, _BLK)], sb, sem)
        pltpu.async_copy(dst_hbm.at[pl.ds(e0, _BLK)], dbufs[m], sem)
        pltpu.async_copy(w_hbm.at[pl.ds(e0, _BLK)], wb, sem)

    def wait_in(m):
        sb, wb, _, sem, _ = sw[m % 2]
        pltpu.make_async_copy(src_hbm.at[pl.ds(0, _BLK)], sb, sem).wait()
        pltpu.make_async_copy(dst_hbm.at[pl.ds(0, _BLK)], dbufs[m],
                              sem).wait()
        pltpu.make_async_copy(w_hbm.at[pl.ds(0, _BLK)], wb, sem).wait()

    def compute(m):
        sb, wb, pb, _, _ = sw[m % 2]
        for i in range(_BLK // _LANES):
            sl = pl.ds(i * _LANES, _LANES)
            xg = plsc.load_gather(x_vmem, [sb[sl]])
            pb[sl] = xg * wb[sl]

    def issue_sc(m):
        _, _, pb, _, sem = sw[m % 2]
        pltpu.async_copy(pb, xmsh.at[dbufs[m]], sem, add=True)

    def wait_sc(m):
        _, _, pb, _, sem = sw[m % 2]
        pltpu.make_async_copy(pb, xmsh.at[dbufs[m]], sem).wait()

    @pl.loop(0, _BLK // _LANES)
    def _(j):
        zbuf[pl.ds(j * _LANES, _LANES)] = _f32x16(0.0)

    for off, ln in _CHUNKS:
        pltpu.sync_copy(zbuf.at[pl.ds(0, ln)],
                        xmsh.at[pl.ds(nbase + off, ln)])
    plsc.subcore_barrier()

    @pl.loop(0, _NBLK)
    def _(blk):
        e0 = ebase + blk * _BLK
        pltpu.sync_copy(src_hbm.at[pl.ds(e0, _BLK)], srcv)
        pltpu.sync_copy(ew_hbm.at[pl.ds(e0, _BLK)], wv)
        pltpu.sync_copy(wv, xmsh.at[srcv], add=True)
    plsc.subcore_barrier()

    pltpu.sync_copy(xmsh, x_vmem)
    plsc.subcore_barrier()

    for off, ln in _CHUNKS:
        pltpu.sync_copy(zbuf.at[pl.ds(0, ln)],
                        xmsh.at[pl.ds(nbase + off, ln)])

    @pl.loop(0, _NBLK)
    def _(blk):
        e0 = ebase + blk * _BLK
        pltpu.sync_copy(src_hbm.at[pl.ds(e0, _BLK)], srcv)
        pltpu.sync_copy(ew_hbm.at[pl.ds(e0, _BLK)], wv)
        for i in range(_BLK // _LANES):
            sl = pl.ds(i * _LANES, _LANES)
            rs = plsc.load_gather(x_vmem, [srcv[sl]])
            nz = rs != _f32x16(0.0)
            safe = jnp.where(nz, rs, _f32x16(1.0))
            pv[sl] = jnp.where(nz, wv[sl] / safe, _f32x16(0.0))
        pltpu.sync_copy(pv, w_hbm.at[pl.ds(e0, _BLK)])

    @pl.when(sid == 0)
    def _():
        lane0 = lax.iota(jnp.int32, _LANES) == 0
        pv[pl.ds(0, _LANES)] = jnp.where(lane0, _f32x16(_WSELF),
                                         _f32x16(_BINIT))
        for k in range(1, _NSRC // _LANES):
            pv[pl.ds(k * _LANES, _LANES)] = _f32x16(_BINIT)
        pv[pl.ds(_NSRC, _LANES)] = jnp.where(lane0, _f32x16(_BINIT),
                                             _f32x16(0.0))
        for k in range(_NSRC // _LANES + 1, _NVIRT // _LANES):
            pv[pl.ds(k * _LANES, _LANES)] = _f32x16(0.0)
        pltpu.sync_copy(pv.at[pl.ds(0, _NVIRT)], w_hbm.at[pl.ds(0, _NVIRT)])

    @pl.loop(0, _NPAD // _LANES)
    def _(j):
        x_vmem[pl.ds(j * _LANES, _LANES)] = _f32x16(1.0 / _N)
    x_vmem[pl.ds(_N, _LANES)] = jnp.where(lax.iota(jnp.int32, _LANES) == 0,
                                          _f32x16(1.0), _f32x16(0.0))
    plsc.subcore_barrier()

    @pl.loop(0, _ITERS)
    def _(it):
        issue_in(0, 0)
        issue_in(1, 1)

        @pl.when(it > 0)
        def _():
            pltpu.make_async_copy(xout_hbm, x_vmem, sem_bc).wait()

        @pl.loop(0, _NBLK // 4)
        def _(g):
            b0 = 4 * g
            for j in range(4):
                wait_in(j)
                compute(j)
                if j == 0:
                    @pl.when(g > 0)
                    def _():
                        wait_sc(3)
                else:
                    wait_sc(j - 1)
                issue_sc(j)

                @pl.when(b0 + j + 2 < _NBLK)
                def _():
                    issue_in(b0 + j + 2, (j + 2) % 4)

        wait_sc(3)
        pltpu.sync_copy(zbuf, xmsh.at[dbufs[3]], add=True)
        plsc.subcore_barrier()

        for off, ln in _CHUNKS:
            pltpu.sync_copy(xmsh.at[pl.ds(nbase + off, ln)],
                            pv.at[pl.ds(0, ln)])

            @pl.loop(0, ln // _LANES)
            def _(j):
                sl = pl.ds(j * _LANES, _LANES)
                pv[sl] = pv[sl] * _ALPHA

            pltpu.sync_copy(pv.at[pl.ds(0, ln)],
                            xout_hbm.at[pl.ds(nbase + off, ln)])
            pltpu.sync_copy(zbuf.at[pl.ds(0, ln)],
                            xmsh.at[pl.ds(nbase + off, ln)])
        plsc.subcore_barrier()
        pltpu.async_copy(xout_hbm, x_vmem, sem_bc)

    pltpu.make_async_copy(xout_hbm, x_vmem, sem_bc).wait()


_cp = pltpu.CompilerParams()
if "needs_layout_passes" in pltpu.CompilerParams.__dataclass_fields__:
    _cp = dataclasses.replace(_cp, needs_layout_passes=False)

_pr_call = functools.partial(
    pl.kernel,
    compiler_params=_cp,
    out_type=(jax.ShapeDtypeStruct((_NPAD,), jnp.float32),
              jax.ShapeDtypeStruct((_EPAD,), jnp.float32)),
    mesh=plsc.VectorSubcoreMesh(core_axis_name="c", subcore_axis_name="s",
                                num_cores=1),
    scratch_types=[
        pltpu.VMEM((_NPAD,), jnp.float32),
        pltpu.VMEM((_BLK,), jnp.int32),
        pltpu.VMEM((_BLK,), jnp.float32),
        pltpu.VMEM((_BLK,), jnp.float32),
        pltpu.VMEM((_BLK,), jnp.int32),
        pltpu.VMEM((_BLK,), jnp.float32),
        pltpu.VMEM((_BLK,), jnp.float32),
        pltpu.VMEM((_BLK,), jnp.int32),
        pltpu.VMEM((_BLK,), jnp.int32),
        pltpu.VMEM((_BLK,), jnp.int32),
        pltpu.VMEM((_BLK,), jnp.int32),
        pltpu.VMEM((_BLK,), jnp.float32),
        pltpu.VMEM_SHARED((_NPAD,), jnp.float32),
        pltpu.SemaphoreType.DMA,
        pltpu.SemaphoreType.DMA,
        pltpu.SemaphoreType.DMA,
        pltpu.SemaphoreType.DMA,
        pltpu.SemaphoreType.DMA,
    ],
)(_body)


def kernel(edge_index, edge_weight, source_nodes):
    src = edge_index[0]
    dst = edge_index[1]
    pad = _EPAD - _E - _NVIRT
    fill = jnp.arange(pad, dtype=jnp.int32) % _N
    vsrc = jnp.full((_NVIRT,), _N, dtype=jnp.int32)
    vdst = jnp.concatenate([
        jnp.full((1,), _N, dtype=jnp.int32),
        source_nodes.astype(jnp.int32),
        jnp.arange(_NVIRT - 1 - _NSRC, dtype=jnp.int32) % _N,
    ])
    vw = jnp.zeros((_NVIRT,), jnp.float32)
    src1 = jnp.concatenate([vsrc, src, fill])
    dst1 = jnp.concatenate([vdst, dst, fill])
    ew1 = jnp.concatenate([vw, edge_weight, jnp.zeros((pad,), jnp.float32)])
    xpad, _ = _pr_call(src1, dst1, ew1)
    return xpad[:_N]
